# baseline (speedup 1.0000x reference)
import functools

import jax
import jax.numpy as jnp
from jax import lax
from jax.experimental import pallas as pl
from jax.experimental.pallas import tpu as pltpu
from jax.experimental.pallas import tpu_sc as plsc

N_NODES = 10000
N_EDGES = 320000
DIM = 128
DAUG = 144
NPAD = 10016
CHUNK = 64
NCHUNKS = 160
EDGES_PER_TILE = CHUNK * NCHUNKS
EPAD = EDGES_PER_TILE * 32
ROWS_PER_TILE = NPAD // 16
BM = 1000

_mesh = plsc.VectorSubcoreMesh(core_axis_name="c", subcore_axis_name="s")


@functools.partial(
    pl.kernel,
    out_type=jax.ShapeDtypeStruct((2 * NPAD, DAUG), jnp.float32),
    mesh=_mesh,
    compiler_params=pltpu.CompilerParams(use_tc_tiling_on_sc=False),
    scratch_types=[
        pltpu.VMEM_SHARED((NPAD, DAUG), jnp.float32),
        pltpu.VMEM((NCHUNKS, CHUNK), jnp.int32),
        pltpu.VMEM((NCHUNKS, CHUNK), jnp.int32),
        pltpu.VMEM((CHUNK, DAUG), jnp.float32),
        pltpu.VMEM((CHUNK, DAUG), jnp.float32),
        pltpu.SemaphoreType.DMA,
        pltpu.SemaphoreType.DMA,
    ],
)
def _segsum_sc(xaug, srcp, dstp, out, acc, sidx, didx, b0, b1, s0, s1):
    c = lax.axis_index("c")
    s = lax.axis_index("s")
    wid = s * 2 + c
    my_base = s * ROWS_PER_TILE

    zeros16 = jnp.zeros((16,), jnp.float32)

    def zrow(r, carry):
        for cc in range(DAUG // 16):
            b0[r, pl.ds(cc * 16, 16)] = zeros16
        return carry

    lax.fori_loop(0, CHUNK, zrow, 0)

    def zacc(j, carry):
        pltpu.sync_copy(b0, acc.at[pl.ds(my_base + j * CHUNK, CHUNK)])
        return carry

    lax.fori_loop(0, ROWS_PER_TILE // CHUNK, zacc, 0)
    zrem = ROWS_PER_TILE % CHUNK
    if zrem:
        pltpu.sync_copy(
            b0.at[pl.ds(0, zrem)],
            acc.at[pl.ds(my_base + ROWS_PER_TILE - zrem, zrem)])
    plsc.subcore_barrier()

    crow = wid * NCHUNKS
    pltpu.sync_copy(srcp.at[pl.ds(crow, NCHUNKS)], sidx)
    pltpu.sync_copy(dstp.at[pl.ds(crow, NCHUNKS)], didx)

    pltpu.async_copy(xaug.at[sidx.at[0]], b0, s0)
    pltpu.async_copy(xaug.at[sidx.at[1]], b1, s1)

    def body(i, carry):
        k = 2 * i
        pltpu.make_async_copy(xaug.at[sidx.at[k]], b0, s0).wait()
        pltpu.sync_copy(b0, acc.at[didx.at[k]], add=True)
        pltpu.async_copy(xaug.at[sidx.at[k + 2]], b0, s0)
        pltpu.make_async_copy(xaug.at[sidx.at[k + 1]], b1, s1).wait()
        pltpu.sync_copy(b1, acc.at[didx.at[k + 1]], add=True)
        pltpu.async_copy(xaug.at[sidx.at[k + 3]], b1, s1)
        return carry

    lax.fori_loop(0, (NCHUNKS - 2) // 2, body, 0)
    pltpu.make_async_copy(xaug.at[sidx.at[NCHUNKS - 2]], b0, s0).wait()
    pltpu.sync_copy(b0, acc.at[didx.at[NCHUNKS - 2]], add=True)
    pltpu.make_async_copy(xaug.at[sidx.at[NCHUNKS - 1]], b1, s1).wait()
    pltpu.sync_copy(b1, acc.at[didx.at[NCHUNKS - 1]], add=True)
    plsc.subcore_barrier()

    pltpu.sync_copy(
        acc.at[pl.ds(my_base, ROWS_PER_TILE)],
        out.at[pl.ds(c * NPAD + my_base, ROWS_PER_TILE)],
    )


def _combine_body(p_ref, x_ref, w_ref, b_ref, f_ref, ssq_ref):
    p = p_ref[0] + p_ref[1]
    hsum = p[:, :DIM]
    deg = jnp.sum(p[:, DIM:], axis=1, keepdims=True)
    deg = jnp.maximum(deg, 1.0)
    h = hsum / deg
    w13 = w_ref[:DIM, :] + w_ref[2 * DIM:, :]
    w2 = w_ref[DIM:2 * DIM, :]
    z = jnp.dot(x_ref[...], w13, preferred_element_type=jnp.float32,
                precision=lax.Precision.HIGHEST)
    z = z + jnp.dot(h, w2, preferred_element_type=jnp.float32,
                    precision=lax.Precision.HIGHEST)
    f = jnp.tanh(z + b_ref[...])
    f_ref[...] = f

    blk = jnp.sum(f * f)[None, None]

    @pl.when(pl.program_id(0) == 0)
    def _():
        ssq_ref[...] = blk

    @pl.when(pl.program_id(0) > 0)
    def _():
        ssq_ref[...] = ssq_ref[...] + blk


def _scale_body(f_ref, ssq_ref, o_ref):
    o_ref[...] = f_ref[...] * lax.rsqrt(ssq_ref[...])


def kernel(x, edge_index, W, b):
    x = x.astype(jnp.float32)
    src = edge_index[0].astype(jnp.int32)
    dst = edge_index[1].astype(jnp.int32)
    pad = EPAD - N_EDGES
    srcp = jnp.concatenate(
        [src, jnp.zeros((pad,), jnp.int32)]).reshape(EPAD // CHUNK, CHUNK)
    dstp = jnp.concatenate(
        [dst, jnp.full((pad,), N_NODES, jnp.int32)]).reshape(EPAD // CHUNK, CHUNK)
    xaug = jnp.concatenate(
        [x, jnp.ones((N_NODES, 1), jnp.float32),
         jnp.zeros((N_NODES, DAUG - DIM - 1), jnp.float32)], axis=1)

    partials = _segsum_sc(xaug, srcp, dstp).reshape(2, NPAD, DAUG)

    grid = N_NODES // BM
    f, ssq = pl.pallas_call(
        _combine_body,
        grid=(grid,),
        in_specs=[
            pl.BlockSpec((2, BM, DAUG), lambda i: (0, i, 0)),
            pl.BlockSpec((BM, DIM), lambda i: (i, 0)),
            pl.BlockSpec((3 * DIM, DIM), lambda i: (0, 0)),
            pl.BlockSpec((1, DIM), lambda i: (0, 0)),
        ],
        out_specs=[
            pl.BlockSpec((BM, DIM), lambda i: (i, 0)),
            pl.BlockSpec((1, 1), lambda i: (0, 0)),
        ],
        out_shape=[
            jax.ShapeDtypeStruct((N_NODES, DIM), jnp.float32),
            jax.ShapeDtypeStruct((1, 1), jnp.float32),
        ],
    )(partials, x, W, b.reshape(1, DIM))

    out = pl.pallas_call(
        _scale_body,
        grid=(grid,),
        in_specs=[
            pl.BlockSpec((BM, DIM), lambda i: (i, 0)),
            pl.BlockSpec((1, 1), lambda i: (0, 0)),
        ],
        out_specs=pl.BlockSpec((BM, DIM), lambda i: (i, 0)),
        out_shape=jax.ShapeDtypeStruct((N_NODES, DIM), jnp.float32),
    )(f, ssq)
    return out


# speedup vs baseline: 2.7683x; 2.7683x over previous
import functools

import jax
import jax.numpy as jnp
from jax import lax
from jax.experimental import pallas as pl
from jax.experimental.pallas import tpu as pltpu
from jax.experimental.pallas import tpu_sc as plsc

N_NODES = 10000
N_EDGES = 320000
DIM = 128
DAUG = 144
NPAD = 10016
CHUNK = 64
NCHUNKS = 160
EDGES_PER_TILE = CHUNK * NCHUNKS
EPAD = EDGES_PER_TILE * 32
ROWS_PER_TILE = NPAD // 16
BM = 1000

_mesh = plsc.VectorSubcoreMesh(core_axis_name="c", subcore_axis_name="s")


@functools.partial(
    pl.kernel,
    out_type=jax.ShapeDtypeStruct((2 * NPAD, DAUG), jnp.float32),
    mesh=_mesh,
    compiler_params=pltpu.CompilerParams(use_tc_tiling_on_sc=False),
    scratch_types=[
        pltpu.VMEM_SHARED((NPAD, DAUG), jnp.float32),
        pltpu.VMEM((NCHUNKS, CHUNK), jnp.int32),
        pltpu.VMEM((NCHUNKS, CHUNK), jnp.int32),
        pltpu.VMEM((CHUNK, DAUG), jnp.float32),
        pltpu.VMEM((CHUNK, DAUG), jnp.float32),
        pltpu.SemaphoreType.DMA,
        pltpu.SemaphoreType.DMA,
    ],
)
def _segsum_sc(xaug, srcp, dstp, out, acc, sidx, didx, b0, b1, s0, s1):
    c = lax.axis_index("c")
    s = lax.axis_index("s")
    wid = s * 2 + c
    my_base = s * ROWS_PER_TILE

    zeros16 = jnp.zeros((16,), jnp.float32)

    def zrow(r, carry):
        for cc in range(DAUG // 16):
            b0[r, pl.ds(cc * 16, 16)] = zeros16
        return carry

    lax.fori_loop(0, CHUNK, zrow, 0)

    def zacc(j, carry):
        pltpu.sync_copy(b0, acc.at[pl.ds(my_base + j * CHUNK, CHUNK)])
        return carry

    lax.fori_loop(0, ROWS_PER_TILE // CHUNK, zacc, 0)
    zrem = ROWS_PER_TILE % CHUNK
    if zrem:
        pltpu.sync_copy(
            b0.at[pl.ds(0, zrem)],
            acc.at[pl.ds(my_base + ROWS_PER_TILE - zrem, zrem)])
    plsc.subcore_barrier()

    crow = wid * NCHUNKS
    pltpu.sync_copy(srcp.at[pl.ds(crow, NCHUNKS)], sidx)
    pltpu.sync_copy(dstp.at[pl.ds(crow, NCHUNKS)], didx)

    pltpu.async_copy(xaug.at[sidx.at[0]], b0, s0)
    pltpu.async_copy(xaug.at[sidx.at[1]], b1, s1)

    def body(i, carry):
        k = 2 * i
        pltpu.make_async_copy(xaug.at[sidx.at[k]], b0, s0).wait()
        pltpu.sync_copy(b0, acc.at[didx.at[k]], add=True)
        pltpu.async_copy(xaug.at[sidx.at[k + 2]], b0, s0)
        pltpu.make_async_copy(xaug.at[sidx.at[k + 1]], b1, s1).wait()
        pltpu.sync_copy(b1, acc.at[didx.at[k + 1]], add=True)
        pltpu.async_copy(xaug.at[sidx.at[k + 3]], b1, s1)
        return carry

    lax.fori_loop(0, (NCHUNKS - 2) // 2, body, 0)
    pltpu.make_async_copy(xaug.at[sidx.at[NCHUNKS - 2]], b0, s0).wait()
    pltpu.sync_copy(b0, acc.at[didx.at[NCHUNKS - 2]], add=True)
    pltpu.make_async_copy(xaug.at[sidx.at[NCHUNKS - 1]], b1, s1).wait()
    pltpu.sync_copy(b1, acc.at[didx.at[NCHUNKS - 1]], add=True)
    plsc.subcore_barrier()

    pltpu.sync_copy(
        acc.at[pl.ds(my_base, ROWS_PER_TILE)],
        out.at[pl.ds(c * NPAD + my_base, ROWS_PER_TILE)],
    )


def _combine_body(p_ref, x_ref, w_ref, b_ref, f_ref, ssq_ref):
    p = p_ref[0] + p_ref[1]
    hsum = p[:, :DIM]
    deg = jnp.sum(p[:, DIM:], axis=1, keepdims=True)
    deg = jnp.maximum(deg, 1.0)
    h = hsum / deg
    w13 = w_ref[:DIM, :] + w_ref[2 * DIM:, :]
    w2 = w_ref[DIM:2 * DIM, :]
    z = jnp.dot(x_ref[...], w13, preferred_element_type=jnp.float32,
                precision=lax.Precision.HIGHEST)
    z = z + jnp.dot(h, w2, preferred_element_type=jnp.float32,
                    precision=lax.Precision.HIGHEST)
    f = jnp.tanh(z + b_ref[...])
    f_ref[...] = f

    blk = jnp.sum(f * f)[None, None]

    @pl.when(pl.program_id(0) == 0)
    def _():
        ssq_ref[...] = blk

    @pl.when(pl.program_id(0) > 0)
    def _():
        ssq_ref[...] = ssq_ref[...] + blk


def _scale_body(f_ref, ssq_ref, o_ref):
    o_ref[...] = f_ref[...] * lax.rsqrt(ssq_ref[...])


def kernel(x, edge_index, W, b):
    x = x.astype(jnp.float32)
    src = edge_index[0].astype(jnp.int32)
    dst = edge_index[1].astype(jnp.int32)
    pad = EPAD - N_EDGES
    cyc = jnp.arange(pad, dtype=jnp.int32) % (NPAD - N_NODES)
    srcp = jnp.concatenate([src, cyc]).reshape(EPAD // CHUNK, CHUNK)
    dstp = jnp.concatenate(
        [dst, N_NODES + cyc]).reshape(EPAD // CHUNK, CHUNK)
    xaug = jnp.concatenate(
        [x, jnp.ones((N_NODES, 1), jnp.float32),
         jnp.zeros((N_NODES, DAUG - DIM - 1), jnp.float32)], axis=1)

    partials = _segsum_sc(xaug, srcp, dstp).reshape(2, NPAD, DAUG)

    grid = N_NODES // BM
    f, ssq = pl.pallas_call(
        _combine_body,
        grid=(grid,),
        in_specs=[
            pl.BlockSpec((2, BM, DAUG), lambda i: (0, i, 0)),
            pl.BlockSpec((BM, DIM), lambda i: (i, 0)),
            pl.BlockSpec((3 * DIM, DIM), lambda i: (0, 0)),
            pl.BlockSpec((1, DIM), lambda i: (0, 0)),
        ],
        out_specs=[
            pl.BlockSpec((BM, DIM), lambda i: (i, 0)),
            pl.BlockSpec((1, 1), lambda i: (0, 0)),
        ],
        out_shape=[
            jax.ShapeDtypeStruct((N_NODES, DIM), jnp.float32),
            jax.ShapeDtypeStruct((1, 1), jnp.float32),
        ],
    )(partials, x, W, b.reshape(1, DIM))

    out = pl.pallas_call(
        _scale_body,
        grid=(grid,),
        in_specs=[
            pl.BlockSpec((BM, DIM), lambda i: (i, 0)),
            pl.BlockSpec((1, 1), lambda i: (0, 0)),
        ],
        out_specs=pl.BlockSpec((BM, DIM), lambda i: (i, 0)),
        out_shape=jax.ShapeDtypeStruct((N_NODES, DIM), jnp.float32),
    )(f, ssq)
    return out


# speedup vs baseline: 2.7867x; 1.0066x over previous
import functools

import jax
import jax.numpy as jnp
from jax import lax
from jax.experimental import pallas as pl
from jax.experimental.pallas import tpu as pltpu
from jax.experimental.pallas import tpu_sc as plsc

N_NODES = 10000
N_EDGES = 320000
DIM = 128
DAUG = 144
NPAD = 10016
CHUNK = 64
NCHUNKS = 160
EDGES_PER_TILE = CHUNK * NCHUNKS
EPAD = EDGES_PER_TILE * 32
ROWS_PER_TILE = NPAD // 16
BM = 1000

_mesh = plsc.VectorSubcoreMesh(core_axis_name="c", subcore_axis_name="s")


@functools.partial(
    pl.kernel,
    out_type=[
        jax.ShapeDtypeStruct((NPAD, DAUG), jnp.float32),
        jax.ShapeDtypeStruct((NPAD, DAUG), jnp.float32),
    ],
    mesh=_mesh,
    compiler_params=pltpu.CompilerParams(use_tc_tiling_on_sc=False),
    scratch_types=[
        pltpu.VMEM_SHARED((NPAD, DAUG), jnp.float32),
        pltpu.VMEM((NCHUNKS, CHUNK), jnp.int32),
        pltpu.VMEM((NCHUNKS, CHUNK), jnp.int32),
        pltpu.VMEM((CHUNK, DAUG), jnp.float32),
        pltpu.VMEM((CHUNK, DAUG), jnp.float32),
        pltpu.SemaphoreType.DMA,
        pltpu.SemaphoreType.DMA,
    ],
)
def _segsum_sc(xaug, srcp, dstp, out0, out1, acc, sidx, didx, b0, b1, s0, s1):
    c = lax.axis_index("c")
    s = lax.axis_index("s")
    wid = s * 2 + c
    my_base = s * ROWS_PER_TILE

    zeros16 = jnp.zeros((16,), jnp.float32)

    def zrow(r, carry):
        for cc in range(DAUG // 16):
            b0[r, pl.ds(cc * 16, 16)] = zeros16
        return carry

    lax.fori_loop(0, CHUNK, zrow, 0)

    def zacc(j, carry):
        pltpu.sync_copy(b0, acc.at[pl.ds(my_base + j * CHUNK, CHUNK)])
        return carry

    lax.fori_loop(0, ROWS_PER_TILE // CHUNK, zacc, 0)
    zrem = ROWS_PER_TILE % CHUNK
    if zrem:
        pltpu.sync_copy(
            b0.at[pl.ds(0, zrem)],
            acc.at[pl.ds(my_base + ROWS_PER_TILE - zrem, zrem)])
    plsc.subcore_barrier()

    crow = wid * NCHUNKS
    pltpu.sync_copy(srcp.at[pl.ds(crow, NCHUNKS)], sidx)
    pltpu.sync_copy(dstp.at[pl.ds(crow, NCHUNKS)], didx)

    pltpu.async_copy(xaug.at[sidx.at[0]], b0, s0)
    pltpu.async_copy(xaug.at[sidx.at[1]], b1, s1)

    def body(i, carry):
        k = 2 * i
        pltpu.make_async_copy(xaug.at[sidx.at[k]], b0, s0).wait()
        pltpu.sync_copy(b0, acc.at[didx.at[k]], add=True)
        pltpu.async_copy(xaug.at[sidx.at[k + 2]], b0, s0)
        pltpu.make_async_copy(xaug.at[sidx.at[k + 1]], b1, s1).wait()
        pltpu.sync_copy(b1, acc.at[didx.at[k + 1]], add=True)
        pltpu.async_copy(xaug.at[sidx.at[k + 3]], b1, s1)
        return carry

    lax.fori_loop(0, (NCHUNKS - 2) // 2, body, 0)
    pltpu.make_async_copy(xaug.at[sidx.at[NCHUNKS - 2]], b0, s0).wait()
    pltpu.sync_copy(b0, acc.at[didx.at[NCHUNKS - 2]], add=True)
    pltpu.make_async_copy(xaug.at[sidx.at[NCHUNKS - 1]], b1, s1).wait()
    pltpu.sync_copy(b1, acc.at[didx.at[NCHUNKS - 1]], add=True)
    plsc.subcore_barrier()

    @pl.when(c == 0)
    def _():
        pltpu.sync_copy(acc.at[pl.ds(my_base, ROWS_PER_TILE)],
                        out0.at[pl.ds(my_base, ROWS_PER_TILE)])

    @pl.when(c == 1)
    def _():
        pltpu.sync_copy(acc.at[pl.ds(my_base, ROWS_PER_TILE)],
                        out1.at[pl.ds(my_base, ROWS_PER_TILE)])


def _combine_body(p0_ref, p1_ref, x_ref, w_ref, b_ref, f_ref, ssq_ref):
    p = p0_ref[...] + p1_ref[...]
    hsum = p[:, :DIM]
    deg = jnp.sum(p[:, DIM:], axis=1, keepdims=True)
    deg = jnp.maximum(deg, 1.0)
    h = hsum / deg
    w13 = w_ref[:DIM, :] + w_ref[2 * DIM:, :]
    w2 = w_ref[DIM:2 * DIM, :]
    z = jnp.dot(x_ref[...], w13, preferred_element_type=jnp.float32,
                precision=lax.Precision.HIGHEST)
    z = z + jnp.dot(h, w2, preferred_element_type=jnp.float32,
                    precision=lax.Precision.HIGHEST)
    f = jnp.tanh(z + b_ref[...])
    f_ref[...] = f

    blk = jnp.sum(f * f)[None, None]

    @pl.when(pl.program_id(0) == 0)
    def _():
        ssq_ref[...] = blk

    @pl.when(pl.program_id(0) > 0)
    def _():
        ssq_ref[...] = ssq_ref[...] + blk


def _scale_body(f_ref, ssq_ref, o_ref):
    o_ref[...] = f_ref[...] * lax.rsqrt(ssq_ref[...])


def kernel(x, edge_index, W, b):
    x = x.astype(jnp.float32)
    src = edge_index[0].astype(jnp.int32)
    dst = edge_index[1].astype(jnp.int32)
    pad = EPAD - N_EDGES
    cyc = jnp.arange(pad, dtype=jnp.int32) % (NPAD - N_NODES)
    srcp = jnp.concatenate([src, cyc]).reshape(EPAD // CHUNK, CHUNK)
    dstp = jnp.concatenate(
        [dst, N_NODES + cyc]).reshape(EPAD // CHUNK, CHUNK)
    xaug = jnp.concatenate(
        [x, jnp.ones((N_NODES, 1), jnp.float32),
         jnp.zeros((N_NODES, DAUG - DIM - 1), jnp.float32)], axis=1)

    p0, p1 = _segsum_sc(xaug, srcp, dstp)

    grid = N_NODES // BM
    f, ssq = pl.pallas_call(
        _combine_body,
        grid=(grid,),
        in_specs=[
            pl.BlockSpec((BM, DAUG), lambda i: (i, 0)),
            pl.BlockSpec((BM, DAUG), lambda i: (i, 0)),
            pl.BlockSpec((BM, DIM), lambda i: (i, 0)),
            pl.BlockSpec((3 * DIM, DIM), lambda i: (0, 0)),
            pl.BlockSpec((1, DIM), lambda i: (0, 0)),
        ],
        out_specs=[
            pl.BlockSpec((BM, DIM), lambda i: (i, 0)),
            pl.BlockSpec((1, 1), lambda i: (0, 0)),
        ],
        out_shape=[
            jax.ShapeDtypeStruct((N_NODES, DIM), jnp.float32),
            jax.ShapeDtypeStruct((1, 1), jnp.float32),
        ],
    )(p0, p1, x, W, b.reshape(1, DIM))

    out = pl.pallas_call(
        _scale_body,
        grid=(grid,),
        in_specs=[
            pl.BlockSpec((BM, DIM), lambda i: (i, 0)),
            pl.BlockSpec((1, 1), lambda i: (0, 0)),
        ],
        out_specs=pl.BlockSpec((BM, DIM), lambda i: (i, 0)),
        out_shape=jax.ShapeDtypeStruct((N_NODES, DIM), jnp.float32),
    )(f, ssq)
    return out


# speedup vs baseline: 3.3808x; 1.2132x over previous
import functools

import jax
import jax.numpy as jnp
from jax import lax
from jax.experimental import pallas as pl
from jax.experimental.pallas import tpu as pltpu
from jax.experimental.pallas import tpu_sc as plsc

N_NODES = 10000
N_EDGES = 320000
DIM = 128
DEGW = 16
NPAD = 10016
CHUNK = 64
EROWS = N_EDGES // CHUNK
NCH = 157
ROWS_PER_TILE = NPAD // 16
BM = 1000

_mesh = plsc.VectorSubcoreMesh(core_axis_name="c", subcore_axis_name="s")


@functools.partial(
    pl.kernel,
    out_type=[
        jax.ShapeDtypeStruct((NPAD, DIM), jnp.float32),
        jax.ShapeDtypeStruct((NPAD, DIM), jnp.float32),
        jax.ShapeDtypeStruct((NPAD, DEGW), jnp.float32),
        jax.ShapeDtypeStruct((NPAD, DEGW), jnp.float32),
    ],
    mesh=_mesh,
    compiler_params=pltpu.CompilerParams(use_tc_tiling_on_sc=False),
    scratch_types=[
        pltpu.VMEM_SHARED((NPAD, DIM), jnp.float32),
        pltpu.VMEM_SHARED((NPAD, DEGW), jnp.float32),
        pltpu.VMEM((NCH, CHUNK), jnp.int32),
        pltpu.VMEM((NCH, CHUNK), jnp.int32),
        pltpu.VMEM((CHUNK, DIM), jnp.float32),
        pltpu.VMEM((CHUNK, DIM), jnp.float32),
        pltpu.VMEM((CHUNK, DEGW), jnp.float32),
        pltpu.SemaphoreType.DMA,
        pltpu.SemaphoreType.DMA,
    ],
)
def _segsum_sc(x, ei, h0, h1, d0, d1, acch, accd, sidx, didx, b0, b1, ones,
               s0, s1):
    c = lax.axis_index("c")
    s = lax.axis_index("s")
    wid = s * 2 + c
    my_base = s * ROWS_PER_TILE

    zeros16 = jnp.zeros((16,), jnp.float32)
    ones16 = jnp.ones((16,), jnp.float32)

    def fill(r, carry):
        for cc in range(DIM // 16):
            b0[r, pl.ds(cc * 16, 16)] = zeros16
        ones[r, :] = zeros16
        return carry

    lax.fori_loop(0, CHUNK, fill, 0)

    def zacc(j, carry):
        pltpu.sync_copy(b0, acch.at[pl.ds(my_base + j * CHUNK, CHUNK)])
        pltpu.sync_copy(ones, accd.at[pl.ds(my_base + j * CHUNK, CHUNK)])
        return carry

    lax.fori_loop(0, ROWS_PER_TILE // CHUNK, zacc, 0)
    zrem = ROWS_PER_TILE % CHUNK
    if zrem:
        pltpu.sync_copy(
            b0.at[pl.ds(0, zrem)],
            acch.at[pl.ds(my_base + ROWS_PER_TILE - zrem, zrem)])
        pltpu.sync_copy(
            ones.at[pl.ds(0, zrem)],
            accd.at[pl.ds(my_base + ROWS_PER_TILE - zrem, zrem)])

    def fillones(r, carry):
        ones[r, :] = ones16
        return carry

    lax.fori_loop(0, CHUNK, fillones, 0)
    plsc.subcore_barrier()

    base_raw = wid * NCH
    nch = jnp.minimum(NCH, EROWS - base_raw)
    base = jnp.minimum(base_raw, EROWS - NCH)
    skew = base_raw - base

    pltpu.sync_copy(ei.at[0].at[pl.ds(base, NCH)], sidx)
    pltpu.sync_copy(ei.at[1].at[pl.ds(base, NCH)], didx)

    pltpu.async_copy(x.at[sidx.at[skew]], b0, s0)
    pltpu.async_copy(x.at[sidx.at[skew + 1]], b1, s1)

    def body(i, carry):
        k = 2 * i
        pltpu.make_async_copy(x.at[sidx.at[skew + k]], b0, s0).wait()
        pltpu.sync_copy(b0, acch.at[didx.at[skew + k]], add=True)
        pltpu.sync_copy(ones, accd.at[didx.at[skew + k]], add=True)

        @pl.when(k + 2 < nch)
        def _():
            pltpu.async_copy(x.at[sidx.at[skew + k + 2]], b0, s0)

        pltpu.make_async_copy(x.at[sidx.at[skew + k + 1]], b1, s1).wait()
        pltpu.sync_copy(b1, acch.at[didx.at[skew + k + 1]], add=True)
        pltpu.sync_copy(ones, accd.at[didx.at[skew + k + 1]], add=True)

        @pl.when(k + 3 < nch)
        def _():
            pltpu.async_copy(x.at[sidx.at[skew + k + 3]], b1, s1)

        return carry

    lax.fori_loop(0, nch // 2, body, 0)

    @pl.when(nch % 2 == 1)
    def _():
        last = skew + nch - 1
        pltpu.make_async_copy(x.at[sidx.at[last]], b0, s0).wait()
        pltpu.sync_copy(b0, acch.at[didx.at[last]], add=True)
        pltpu.sync_copy(ones, accd.at[didx.at[last]], add=True)

    plsc.subcore_barrier()

    @pl.when(c == 0)
    def _():
        pltpu.sync_copy(acch.at[pl.ds(my_base, ROWS_PER_TILE)],
                        h0.at[pl.ds(my_base, ROWS_PER_TILE)])
        pltpu.sync_copy(accd.at[pl.ds(my_base, ROWS_PER_TILE)],
                        d0.at[pl.ds(my_base, ROWS_PER_TILE)])

    @pl.when(c == 1)
    def _():
        pltpu.sync_copy(acch.at[pl.ds(my_base, ROWS_PER_TILE)],
                        h1.at[pl.ds(my_base, ROWS_PER_TILE)])
        pltpu.sync_copy(accd.at[pl.ds(my_base, ROWS_PER_TILE)],
                        d1.at[pl.ds(my_base, ROWS_PER_TILE)])


def _combine_body(p0_ref, p1_ref, d0_ref, d1_ref, x_ref, w_ref, b_ref,
                  f_ref, ssq_ref):
    hsum = p0_ref[...] + p1_ref[...]
    deg = jnp.sum(d0_ref[...] + d1_ref[...], axis=1, keepdims=True)
    deg = jnp.maximum(deg, 1.0)
    h = hsum / deg
    w13 = w_ref[:DIM, :] + w_ref[2 * DIM:, :]
    w2 = w_ref[DIM:2 * DIM, :]
    z = jnp.dot(x_ref[...], w13, preferred_element_type=jnp.float32,
                precision=lax.Precision.HIGHEST)
    z = z + jnp.dot(h, w2, preferred_element_type=jnp.float32,
                    precision=lax.Precision.HIGHEST)
    f = jnp.tanh(z + b_ref[...])
    f_ref[...] = f

    blk = jnp.sum(f * f)[None, None]

    @pl.when(pl.program_id(0) == 0)
    def _():
        ssq_ref[...] = blk

    @pl.when(pl.program_id(0) > 0)
    def _():
        ssq_ref[...] = ssq_ref[...] + blk


def _scale_body(f_ref, ssq_ref, o_ref):
    o_ref[...] = f_ref[...] * lax.rsqrt(ssq_ref[...])


def kernel(x, edge_index, W, b):
    x = x.astype(jnp.float32)
    ei = edge_index.astype(jnp.int32).reshape(2, EROWS, CHUNK)

    p0, p1, d0, d1 = _segsum_sc(x, ei)

    grid = N_NODES // BM
    f, ssq = pl.pallas_call(
        _combine_body,
        grid=(grid,),
        in_specs=[
            pl.BlockSpec((BM, DIM), lambda i: (i, 0)),
            pl.BlockSpec((BM, DIM), lambda i: (i, 0)),
            pl.BlockSpec((BM, DEGW), lambda i: (i, 0)),
            pl.BlockSpec((BM, DEGW), lambda i: (i, 0)),
            pl.BlockSpec((BM, DIM), lambda i: (i, 0)),
            pl.BlockSpec((3 * DIM, DIM), lambda i: (0, 0)),
            pl.BlockSpec((1, DIM), lambda i: (0, 0)),
        ],
        out_specs=[
            pl.BlockSpec((BM, DIM), lambda i: (i, 0)),
            pl.BlockSpec((1, 1), lambda i: (0, 0)),
        ],
        out_shape=[
            jax.ShapeDtypeStruct((N_NODES, DIM), jnp.float32),
            jax.ShapeDtypeStruct((1, 1), jnp.float32),
        ],
    )(p0, p1, d0, d1, x, W, b.reshape(1, DIM))

    out = pl.pallas_call(
        _scale_body,
        grid=(grid,),
        in_specs=[
            pl.BlockSpec((BM, DIM), lambda i: (i, 0)),
            pl.BlockSpec((1, 1), lambda i: (0, 0)),
        ],
        out_specs=pl.BlockSpec((BM, DIM), lambda i: (i, 0)),
        out_shape=jax.ShapeDtypeStruct((N_NODES, DIM), jnp.float32),
    )(f, ssq)
    return out
